# Initial kernel scaffold; baseline (speedup 1.0000x reference)
import jax
import jax.numpy as jnp
from jax.experimental import pallas as pl


def kernel(t, f0, index, v0, epsilon_v, epsilon_r):
    raise NotImplementedError("write your pallas kernel here")


# speedup vs baseline: 2.9759x; 2.9759x over previous
import functools

import jax
import jax.numpy as jnp
from jax import lax
from jax.experimental import pallas as pl
from jax.experimental.pallas import tpu as pltpu
from jax.experimental.pallas import tpu_sc as plsc

N = 262144
S = 2048
S3 = 3 * S
NC = 2
NS = 16
NW = NC * NS
CH = N // NW
CH3 = 3 * CH
SCH = CH // 2
SCH3 = 3 * SCH
EPS = 1e-5

_MESH = dict(core_axis_name="c", subcore_axis_name="s", num_cores=NC,
             num_subcores=NS)


def _lane():
    return lax.iota(jnp.int32, 16)


def _fill(ref, n, val):
    v = jnp.full((16,), val, ref.dtype)

    @pl.loop(0, n // 16)
    def _(i):
        ref[pl.ds(i * 16, 16)] = v


def _sqrt(x):
    i = plsc.bitcast(x, jnp.int32)
    i = jnp.int32(0x5F3759DF) - (i >> 1)
    y = plsc.bitcast(i, jnp.float32)
    for _ in range(3):
        y = y * (1.5 - 0.5 * x * y * y)
    return x * y


def _floor(y):
    t = y.astype(jnp.int32).astype(jnp.float32)
    return t - jnp.where(y < t, 1.0, 0.0)


def _wrap_disp(x):
    y = x + 0.5
    return y - _floor(y) - 0.5


def _wrap_pos(x):
    return x - _floor(x)


def _build_idx3(idx_ref, idx3_ref, nvec):
    lane = _lane()

    @pl.loop(0, nvec)
    def _(i):
        s3 = idx_ref[pl.ds(i * 16, 16)] * 3
        base = i * 48
        for c in range(3):
            plsc.store_scatter(idx3_ref, [base + lane * 3 + c], s3 + c)


def _combine_counts(ct_hbm, cts_ref, tmp3_ref):
    pltpu.sync_copy(ct_hbm.at[0], cts_ref)
    pltpu.sync_copy(ct_hbm.at[1], tmp3_ref.at[pl.ds(0, S)])

    @pl.loop(0, S // 16)
    def _(i):
        sl = pl.ds(i * 16, 16)
        cts_ref[sl] = jnp.maximum(cts_ref[sl] + tmp3_ref[sl], 1.0)


def _mean_table(p_hbm, cts_ref, mean_ref, tmp3_ref):
    pltpu.sync_copy(p_hbm.at[0], mean_ref)
    pltpu.sync_copy(p_hbm.at[1], tmp3_ref)
    lane = _lane()

    @pl.loop(0, S3 // 16)
    def _(i):
        sl = pl.ds(i * 16, 16)
        c = plsc.load_gather(cts_ref, [(i * 16 + lane) // 3])
        mean_ref[sl] = (mean_ref[sl] + tmp3_ref[sl]) / c


def _seg_partials_body(idx_hbm, ev_hbm, er_hbm, sv_out, sr_out, ct_out,
                       idx_v, idx3_v, val_v, ones_v, zeros_v, av, ar, ac):
    cid = lax.axis_index("c")
    sid = lax.axis_index("s")
    wid = sid * NC + cid
    base = wid * CH
    _fill(ones_v, CH, 1.0)
    _fill(zeros_v, S3, 0.0)

    @pl.when(sid == 0)
    def _():
        pltpu.sync_copy(zeros_v, av)
        pltpu.sync_copy(zeros_v, ar)
        pltpu.sync_copy(zeros_v.at[pl.ds(0, S)], ac)

    plsc.subcore_barrier()
    pltpu.sync_copy(idx_hbm.at[pl.ds(base, CH)], idx_v)
    _build_idx3(idx_v, idx3_v, CH // 16)
    pltpu.sync_copy(ev_hbm.at[pl.ds(base * 3, CH3)], val_v)
    pltpu.sync_copy(val_v, av.at[idx3_v], add=True)
    pltpu.sync_copy(er_hbm.at[pl.ds(base * 3, CH3)], val_v)
    pltpu.sync_copy(val_v, ar.at[idx3_v], add=True)
    pltpu.sync_copy(ones_v, ac.at[idx_v], add=True)
    plsc.subcore_barrier()

    @pl.when(sid == 0)
    def _():
        pltpu.sync_copy(av, sv_out.at[cid])
        pltpu.sync_copy(ar, sr_out.at[cid])
        pltpu.sync_copy(ac, ct_out.at[cid])


_CP = pltpu.CompilerParams(needs_layout_passes=False)

_seg_partials = functools.partial(
    pl.kernel,
    compiler_params=_CP,
    out_type=(jax.ShapeDtypeStruct((NC, S3), jnp.float32),
              jax.ShapeDtypeStruct((NC, S3), jnp.float32),
              jax.ShapeDtypeStruct((NC, S), jnp.float32)),
    mesh=plsc.VectorSubcoreMesh(**_MESH),
    scratch_types=[
        pltpu.VMEM((CH,), jnp.int32),
        pltpu.VMEM((CH3,), jnp.int32),
        pltpu.VMEM((CH3,), jnp.float32),
        pltpu.VMEM((CH,), jnp.float32),
        pltpu.VMEM((S3,), jnp.float32),
        pltpu.VMEM_SHARED((S3,), jnp.float32),
        pltpu.VMEM_SHARED((S3,), jnp.float32),
        pltpu.VMEM_SHARED((S,), jnp.float32),
    ],
)(_seg_partials_body)


def _center_diffuse_body(idx_hbm, t_hbm, v0_hbm, ev_hbm, er_hbm,
                         sv_hbm, sr_hbm, ct_hbm,
                         evc_out, erc_out, vt_out, rp_out, srt_out,
                         idx_v, idx3_v, t_v, a_v, s1_v, co_v, s2_v,
                         ev_v, er_v, v0_v, rp_v, mv_v, mr_v, cts_v, tmp3_v,
                         acc_s):
    cid = lax.axis_index("c")
    sid = lax.axis_index("s")
    wid = sid * NC + cid
    lane = _lane()

    _fill(tmp3_v, S3, 0.0)

    @pl.when(sid == 0)
    def _():
        pltpu.sync_copy(tmp3_v, acc_s)

    _combine_counts(ct_hbm, cts_v, tmp3_v)
    _mean_table(sv_hbm, cts_v, mv_v, tmp3_v)
    _mean_table(sr_hbm, cts_v, mr_v, tmp3_v)
    plsc.subcore_barrier()

    for sub in range(CH // SCH):
        base = wid * CH + sub * SCH
        pltpu.sync_copy(idx_hbm.at[pl.ds(base, SCH)], idx_v)
        pltpu.sync_copy(t_hbm.at[pl.ds(base, SCH)], t_v)
        pltpu.sync_copy(v0_hbm.at[pl.ds(base * 3, SCH3)], v0_v)
        pltpu.sync_copy(ev_hbm.at[pl.ds(base * 3, SCH3)], ev_v)
        pltpu.sync_copy(er_hbm.at[pl.ds(base * 3, SCH3)], er_v)
        _build_idx3(idx_v, idx3_v, SCH // 16)

        @pl.loop(0, SCH // 16)
        def _(i):
            sl = pl.ds(i * 16, 16)
            tt = t_v[sl] * 2.0
            a = jnp.exp(-tt)
            a_v[sl] = a
            s1_v[sl] = _sqrt(jnp.maximum(1.0 - a * a, EPS))
            co_v[sl] = (1.0 - a) / (1.0 + a)
            s2_v[sl] = _sqrt(jnp.maximum(tt + tt + 8.0 * a / (1.0 + a) - 4.0,
                                         EPS))

        @pl.loop(0, SCH3 // 16)
        def _(j):
            sl = pl.ds(j * 16, 16)
            e = (j * 16 + lane) // 3
            i3 = idx3_v[sl]
            gmv = plsc.load_gather(mv_v, [i3])
            gmr = plsc.load_gather(mr_v, [i3])
            af = plsc.load_gather(a_v, [e])
            s1f = plsc.load_gather(s1_v, [e])
            cof = plsc.load_gather(co_v, [e])
            s2f = plsc.load_gather(s2_v, [e])
            evc = ev_v[sl] - gmv
            erc = er_v[sl] - gmr
            v0 = v0_v[sl]
            vt = af * v0 + s1f * evc
            q = cof * (vt + v0) + s2f * erc
            ev_v[sl] = evc
            er_v[sl] = erc
            v0_v[sl] = vt
            rp_v[sl] = _wrap_disp(q)

        pltpu.sync_copy(rp_v, acc_s.at[idx3_v], add=True)
        pltpu.sync_copy(ev_v, evc_out.at[pl.ds(base * 3, SCH3)])
        pltpu.sync_copy(er_v, erc_out.at[pl.ds(base * 3, SCH3)])
        pltpu.sync_copy(v0_v, vt_out.at[pl.ds(base * 3, SCH3)])
        pltpu.sync_copy(rp_v, rp_out.at[pl.ds(base * 3, SCH3)])

    plsc.subcore_barrier()

    @pl.when(sid == 0)
    def _():
        pltpu.sync_copy(acc_s, srt_out.at[cid])


_center_diffuse = functools.partial(
    pl.kernel,
    compiler_params=_CP,
    out_type=(jax.ShapeDtypeStruct((N * 3,), jnp.float32),
              jax.ShapeDtypeStruct((N * 3,), jnp.float32),
              jax.ShapeDtypeStruct((N * 3,), jnp.float32),
              jax.ShapeDtypeStruct((N * 3,), jnp.float32),
              jax.ShapeDtypeStruct((NC, S3), jnp.float32)),
    mesh=plsc.VectorSubcoreMesh(**_MESH),
    scratch_types=[
        pltpu.VMEM((SCH,), jnp.int32),
        pltpu.VMEM((SCH3,), jnp.int32),
        pltpu.VMEM((SCH,), jnp.float32),
        pltpu.VMEM((SCH,), jnp.float32),
        pltpu.VMEM((SCH,), jnp.float32),
        pltpu.VMEM((SCH,), jnp.float32),
        pltpu.VMEM((SCH,), jnp.float32),
        pltpu.VMEM((SCH3,), jnp.float32),
        pltpu.VMEM((SCH3,), jnp.float32),
        pltpu.VMEM((SCH3,), jnp.float32),
        pltpu.VMEM((SCH3,), jnp.float32),
        pltpu.VMEM((S3,), jnp.float32),
        pltpu.VMEM((S3,), jnp.float32),
        pltpu.VMEM((S,), jnp.float32),
        pltpu.VMEM((S3,), jnp.float32),
        pltpu.VMEM_SHARED((S3,), jnp.float32),
    ],
)(_center_diffuse_body)


def _finish_body(idx_hbm, rp_hbm, f0_hbm, srt_hbm, ct_hbm,
                 ft_out, rt_out,
                 idx_v, idx3_v, rp_v, f0_v, mt_v, cts_v, tmp3_v):
    cid = lax.axis_index("c")
    sid = lax.axis_index("s")
    wid = sid * NC + cid
    base = wid * CH

    _combine_counts(ct_hbm, cts_v, tmp3_v)
    _mean_table(srt_hbm, cts_v, mt_v, tmp3_v)

    pltpu.sync_copy(idx_hbm.at[pl.ds(base, CH)], idx_v)
    pltpu.sync_copy(rp_hbm.at[pl.ds(base * 3, CH3)], rp_v)
    pltpu.sync_copy(f0_hbm.at[pl.ds(base * 3, CH3)], f0_v)
    _build_idx3(idx_v, idx3_v, CH // 16)

    @pl.loop(0, CH3 // 16)
    def _(j):
        sl = pl.ds(j * 16, 16)
        g = plsc.load_gather(mt_v, [idx3_v[sl]])
        rt = _wrap_disp(rp_v[sl] - g)
        ft = _wrap_pos(f0_v[sl] + rt)
        rp_v[sl] = rt
        f0_v[sl] = ft

    pltpu.sync_copy(rp_v, rt_out.at[pl.ds(base * 3, CH3)])
    pltpu.sync_copy(f0_v, ft_out.at[pl.ds(base * 3, CH3)])


_finish = functools.partial(
    pl.kernel,
    compiler_params=_CP,
    out_type=(jax.ShapeDtypeStruct((N * 3,), jnp.float32),
              jax.ShapeDtypeStruct((N * 3,), jnp.float32)),
    mesh=plsc.VectorSubcoreMesh(**_MESH),
    scratch_types=[
        pltpu.VMEM((CH,), jnp.int32),
        pltpu.VMEM((CH3,), jnp.int32),
        pltpu.VMEM((CH3,), jnp.float32),
        pltpu.VMEM((CH3,), jnp.float32),
        pltpu.VMEM((S3,), jnp.float32),
        pltpu.VMEM((S,), jnp.float32),
        pltpu.VMEM((S3,), jnp.float32),
    ],
)(_finish_body)


def kernel(t, f0, index, v0, epsilon_v, epsilon_r):
    idx = index.astype(jnp.int32)
    evf = epsilon_v.reshape(N * 3)
    erf = epsilon_r.reshape(N * 3)
    v0f = v0.reshape(N * 3)
    f0f = f0.reshape(N * 3)
    sv, sr, ct = _seg_partials(idx, evf, erf)
    evc, erc, vt, rp, srt = _center_diffuse(idx, t, v0f, evf, erf, sv, sr, ct)
    ft, rt = _finish(idx, rp, f0f, srt, ct)
    rs = lambda x: x.reshape(N, 3)
    return (rs(ft), rs(vt), rs(evc), rs(erc), rs(rt))


# speedup vs baseline: 14.3954x; 4.8373x over previous
import functools

import jax
import jax.numpy as jnp
from jax import lax
from jax.experimental import pallas as pl
from jax.experimental.pallas import tpu as pltpu
from jax.experimental.pallas import tpu_sc as plsc

N = 262144
S = 2048
NC = 2
NS = 16
NW = NC * NS
CH = N // NW
SCH = CH // 2
EPS = 1e-5

_MESH = dict(core_axis_name="c", subcore_axis_name="s", num_cores=NC,
             num_subcores=NS)
_CP = pltpu.CompilerParams(needs_layout_passes=False)


def _fill(ref, n, val):
    v = jnp.full((16,), val, ref.dtype)

    @pl.loop(0, n // 16)
    def _(i):
        ref[pl.ds(i * 16, 16)] = v


def _sqrt(x):
    i = plsc.bitcast(x, jnp.int32)
    i = jnp.int32(0x5F3759DF) - (i >> 1)
    y = plsc.bitcast(i, jnp.float32)
    for _ in range(3):
        y = y * (1.5 - 0.5 * x * y * y)
    return x * y


def _floor(y):
    t = y.astype(jnp.int32).astype(jnp.float32)
    return t - jnp.where(y < t, 1.0, 0.0)


def _wrap_disp(x):
    y = x + 0.5
    return y - _floor(y) - 0.5


def _wrap_pos(x):
    return x - _floor(x)


def _combine_counts(ct_hbm, cts_ref, tmp_ref):
    pltpu.sync_copy(ct_hbm.at[0], cts_ref)
    pltpu.sync_copy(ct_hbm.at[1], tmp_ref)

    @pl.loop(0, S // 16)
    def _(i):
        sl = pl.ds(i * 16, 16)
        cts_ref[sl] = jnp.maximum(cts_ref[sl] + tmp_ref[sl], 1.0)


def _mean_table(p_hbm, row, cts_ref, mean_ref, tmp_ref):
    pltpu.sync_copy(p_hbm.at[row], mean_ref)
    pltpu.sync_copy(p_hbm.at[3 + row], tmp_ref)

    @pl.loop(0, S // 16)
    def _(i):
        sl = pl.ds(i * 16, 16)
        mean_ref[sl] = (mean_ref[sl] + tmp_ref[sl]) / cts_ref[sl]


def _seg_partials_body(idx_hbm, ev_hbm, er_hbm, sv_out, sr_out, ct_out,
                       idx_v, val_v, ones_v, zeros_v,
                       a0, a1, a2, b0, b1, b2, ac):
    cid = lax.axis_index("c")
    sid = lax.axis_index("s")
    wid = sid * NC + cid
    base = wid * CH
    _fill(ones_v, CH, 1.0)
    _fill(zeros_v, S, 0.0)

    accs = (a0, a1, a2, b0, b1, b2, ac)

    @pl.when(sid == 0)
    def _():
        for acc in accs:
            pltpu.sync_copy(zeros_v, acc)

    plsc.subcore_barrier()
    pltpu.sync_copy(idx_hbm.at[pl.ds(base, CH)], idx_v)
    for c, acc in enumerate((a0, a1, a2)):
        pltpu.sync_copy(ev_hbm.at[pl.ds(c * N + base, CH)], val_v)
        pltpu.sync_copy(val_v, acc.at[idx_v], add=True)
    for c, acc in enumerate((b0, b1, b2)):
        pltpu.sync_copy(er_hbm.at[pl.ds(c * N + base, CH)], val_v)
        pltpu.sync_copy(val_v, acc.at[idx_v], add=True)
    pltpu.sync_copy(ones_v, ac.at[idx_v], add=True)
    plsc.subcore_barrier()

    @pl.when(sid == 0)
    def _():
        for c, acc in enumerate((a0, a1, a2)):
            pltpu.sync_copy(acc, sv_out.at[cid * 3 + c])
        for c, acc in enumerate((b0, b1, b2)):
            pltpu.sync_copy(acc, sr_out.at[cid * 3 + c])
        pltpu.sync_copy(ac, ct_out.at[cid])


_seg_partials = functools.partial(
    pl.kernel,
    compiler_params=_CP,
    out_type=(jax.ShapeDtypeStruct((NC * 3, S), jnp.float32),
              jax.ShapeDtypeStruct((NC * 3, S), jnp.float32),
              jax.ShapeDtypeStruct((NC, S), jnp.float32)),
    mesh=plsc.VectorSubcoreMesh(**_MESH),
    scratch_types=[
        pltpu.VMEM((CH,), jnp.int32),
        pltpu.VMEM((CH,), jnp.float32),
        pltpu.VMEM((CH,), jnp.float32),
        pltpu.VMEM((S,), jnp.float32),
    ] + [pltpu.VMEM_SHARED((S,), jnp.float32)] * 7,
)(_seg_partials_body)


def _center_diffuse_body(idx_hbm, t_hbm, v0_hbm, ev_hbm, er_hbm,
                         sv_hbm, sr_hbm, ct_hbm,
                         evc_out, erc_out, vt_out, rp_out, srt_out,
                         idx_v, t_v, a_v, s1_v, co_v, s2_v,
                         ev0, ev1, ev2, er0, er1, er2,
                         v00, v01, v02, rp0, rp1, rp2,
                         mv0, mv1, mv2, mr0, mr1, mr2, cts_v, tmp_v,
                         r0, r1, r2):
    cid = lax.axis_index("c")
    sid = lax.axis_index("s")
    wid = sid * NC + cid

    evs, ers, v0s, rps = (ev0, ev1, ev2), (er0, er1, er2), \
        (v00, v01, v02), (rp0, rp1, rp2)
    mvs, mrs, raccs = (mv0, mv1, mv2), (mr0, mr1, mr2), (r0, r1, r2)

    _fill(tmp_v, S, 0.0)

    @pl.when(sid == 0)
    def _():
        for acc in raccs:
            pltpu.sync_copy(tmp_v, acc)

    _combine_counts(ct_hbm, cts_v, tmp_v)
    for c in range(3):
        _mean_table(sv_hbm, c, cts_v, mvs[c], tmp_v)
        _mean_table(sr_hbm, c, cts_v, mrs[c], tmp_v)
    plsc.subcore_barrier()

    for sub in range(CH // SCH):
        base = wid * CH + sub * SCH
        pltpu.sync_copy(idx_hbm.at[pl.ds(base, SCH)], idx_v)
        pltpu.sync_copy(t_hbm.at[pl.ds(base, SCH)], t_v)
        for c in range(3):
            pltpu.sync_copy(v0_hbm.at[pl.ds(c * N + base, SCH)], v0s[c])
            pltpu.sync_copy(ev_hbm.at[pl.ds(c * N + base, SCH)], evs[c])
            pltpu.sync_copy(er_hbm.at[pl.ds(c * N + base, SCH)], ers[c])

        @pl.loop(0, SCH // 16)
        def _(i):
            sl = pl.ds(i * 16, 16)
            tt = t_v[sl] * 2.0
            a = jnp.exp(-tt)
            a_v[sl] = a
            s1_v[sl] = _sqrt(jnp.maximum(1.0 - a * a, EPS))
            co_v[sl] = (1.0 - a) / (1.0 + a)
            s2_v[sl] = _sqrt(jnp.maximum(tt + tt + 8.0 * a / (1.0 + a) - 4.0,
                                         EPS))

        @pl.loop(0, SCH // 16)
        def _(i):
            sl = pl.ds(i * 16, 16)
            seg = idx_v[sl]
            a = a_v[sl]
            s1 = s1_v[sl]
            co = co_v[sl]
            s2 = s2_v[sl]
            for c in range(3):
                m1 = plsc.load_gather(mvs[c], [seg])
                m2 = plsc.load_gather(mrs[c], [seg])
                evc = evs[c][sl] - m1
                erc = ers[c][sl] - m2
                v0 = v0s[c][sl]
                vt = a * v0 + s1 * evc
                q = co * (vt + v0) + s2 * erc
                evs[c][sl] = evc
                ers[c][sl] = erc
                v0s[c][sl] = vt
                rps[c][sl] = _wrap_disp(q)

        for c in range(3):
            pltpu.sync_copy(rps[c], raccs[c].at[idx_v], add=True)
            pltpu.sync_copy(evs[c], evc_out.at[pl.ds(c * N + base, SCH)])
            pltpu.sync_copy(ers[c], erc_out.at[pl.ds(c * N + base, SCH)])
            pltpu.sync_copy(v0s[c], vt_out.at[pl.ds(c * N + base, SCH)])
            pltpu.sync_copy(rps[c], rp_out.at[pl.ds(c * N + base, SCH)])

    plsc.subcore_barrier()

    @pl.when(sid == 0)
    def _():
        for c in range(3):
            pltpu.sync_copy(raccs[c], srt_out.at[cid * 3 + c])


_center_diffuse = functools.partial(
    pl.kernel,
    compiler_params=_CP,
    out_type=(jax.ShapeDtypeStruct((N * 3,), jnp.float32),
              jax.ShapeDtypeStruct((N * 3,), jnp.float32),
              jax.ShapeDtypeStruct((N * 3,), jnp.float32),
              jax.ShapeDtypeStruct((N * 3,), jnp.float32),
              jax.ShapeDtypeStruct((NC * 3, S), jnp.float32)),
    mesh=plsc.VectorSubcoreMesh(**_MESH),
    scratch_types=[
        pltpu.VMEM((SCH,), jnp.int32),
        pltpu.VMEM((SCH,), jnp.float32),
        pltpu.VMEM((SCH,), jnp.float32),
        pltpu.VMEM((SCH,), jnp.float32),
        pltpu.VMEM((SCH,), jnp.float32),
        pltpu.VMEM((SCH,), jnp.float32),
    ] + [pltpu.VMEM((SCH,), jnp.float32)] * 12
      + [pltpu.VMEM((S,), jnp.float32)] * 8
      + [pltpu.VMEM_SHARED((S,), jnp.float32)] * 3,
)(_center_diffuse_body)


def _finish_body(idx_hbm, rp_hbm, f0_hbm, srt_hbm, ct_hbm,
                 ft_out, rt_out,
                 idx_v, rp_v, f0_v, mt0, mt1, mt2, cts_v, tmp_v):
    cid = lax.axis_index("c")
    sid = lax.axis_index("s")
    wid = sid * NC + cid
    base = wid * CH
    mts = (mt0, mt1, mt2)

    _combine_counts(ct_hbm, cts_v, tmp_v)
    for c in range(3):
        _mean_table(srt_hbm, c, cts_v, mts[c], tmp_v)

    pltpu.sync_copy(idx_hbm.at[pl.ds(base, CH)], idx_v)
    for c in range(3):
        pltpu.sync_copy(rp_hbm.at[pl.ds(c * N + base, CH)], rp_v)
        pltpu.sync_copy(f0_hbm.at[pl.ds(c * N + base, CH)], f0_v)

        @pl.loop(0, CH // 16)
        def _(i):
            sl = pl.ds(i * 16, 16)
            g = plsc.load_gather(mts[c], [idx_v[sl]])
            rt = _wrap_disp(rp_v[sl] - g)
            ft = _wrap_pos(f0_v[sl] + rt)
            rp_v[sl] = rt
            f0_v[sl] = ft

        pltpu.sync_copy(rp_v, rt_out.at[pl.ds(c * N + base, CH)])
        pltpu.sync_copy(f0_v, ft_out.at[pl.ds(c * N + base, CH)])


_finish = functools.partial(
    pl.kernel,
    compiler_params=_CP,
    out_type=(jax.ShapeDtypeStruct((N * 3,), jnp.float32),
              jax.ShapeDtypeStruct((N * 3,), jnp.float32)),
    mesh=plsc.VectorSubcoreMesh(**_MESH),
    scratch_types=[
        pltpu.VMEM((CH,), jnp.int32),
        pltpu.VMEM((CH,), jnp.float32),
        pltpu.VMEM((CH,), jnp.float32),
    ] + [pltpu.VMEM((S,), jnp.float32)] * 5,
)(_finish_body)


def kernel(t, f0, index, v0, epsilon_v, epsilon_r):
    idx = index.astype(jnp.int32)
    planar = lambda x: x.T.reshape(N * 3)
    evf = planar(epsilon_v)
    erf = planar(epsilon_r)
    v0f = planar(v0)
    f0f = planar(f0)
    sv, sr, ct = _seg_partials(idx, evf, erf)
    evc, erc, vt, rp, srt = _center_diffuse(idx, t, v0f, evf, erf, sv, sr, ct)
    ft, rt = _finish(idx, rp, f0f, srt, ct)
    unplanar = lambda x: x.reshape(3, N).T
    return (unplanar(ft), unplanar(vt), unplanar(evc), unplanar(erc),
            unplanar(rt))
